# baseline (speedup 1.0000x reference)
import functools

import jax
import jax.numpy as jnp
from jax import lax
from jax.experimental import pallas as pl
from jax.experimental.pallas import tpu as pltpu
from jax.experimental.pallas import tpu_sc as plsc

N = 10000
E = 160000
D = 256
G = 64
C = 10

NCORES = 2
NSUB = 16
NT = NCORES * NSUB
K = 128
E_PAD = 163840
UNIT = 4096
UNITS = E_PAD // UNIT
M0 = 12
NP = 10112
RPS = NP // NSUB


def _sc_edge(hp, src, dst, aa, wp):
    mesh = plsc.VectorSubcoreMesh(core_axis_name="c", subcore_axis_name="s",
                                  num_cores=NCORES, num_subcores=NSUB)

    def body(hp_hbm, src_hbm, dst_hbm, aa_hbm, out_hbm,
             src0, dst0, src1, dst1, ex0, ex1, asb0, adb0, asb1, adb1,
             rows0, rows1, as_sh, ad_sh, acc,
             semi0, semi1, sema0, sema1, semg0, semg1):
        c = lax.axis_index("c")
        s = lax.axis_index("s")
        bufs = ((src0, dst0, ex0, asb0, adb0, rows0, semi0, sema0, semg0),
                (src1, dst1, ex1, asb1, adb1, rows1, semi1, sema1, semg1))
        pairs = jnp.where(c == 0, M0, UNITS - M0)
        base = jnp.where(c == 0, 0, M0 * UNIT) + s * (2 * K * pairs)

        def zrow(r, carry):
            for cg in range(wp // 16):
                rows0[r, pl.ds(cg * 16, 16)] = jnp.zeros((16,), jnp.float32)
            return carry
        lax.fori_loop(0, K, zrow, 0)
        row0_ = s * RPS
        done = 0
        while done < RPS:
            sz = min(K, RPS - done)
            pltpu.sync_copy(rows0.at[pl.ds(0, sz)],
                            acc.at[pl.ds(row0_ + done, sz)])
            done += sz

        @pl.when(s == 0)
        def _stage_aa():
            pltpu.sync_copy(aa_hbm.at[0], as_sh)
            pltpu.sync_copy(aa_hbm.at[1], ad_sh)
        plsc.subcore_barrier()

        for b in (0, 1):
            srcb, dstb = bufs[b][0], bufs[b][1]
            semi = bufs[b][6]
            off = base + b * K
            pltpu.make_async_copy(src_hbm.at[pl.ds(off, K)], srcb, semi).start()
            pltpu.make_async_copy(dst_hbm.at[pl.ds(off, K)], dstb, semi).start()

        def do_pair(i, issue_next):
            for b in (0, 1):
                srcb, dstb, exb, asb, adb, rows, semi, sema, semg = bufs[b]
                pltpu.make_async_copy(src_hbm.at[pl.ds(0, K)], srcb, semi).wait()
                pltpu.make_async_copy(dst_hbm.at[pl.ds(0, K)], dstb, semi).wait()
                pltpu.make_async_copy(as_sh.at[srcb], asb, sema).start()
                pltpu.make_async_copy(ad_sh.at[dstb], adb, sema).start()
                pltpu.make_async_copy(hp_hbm.at[srcb], rows, semg).start()
            for b in (0, 1):
                srcb, dstb, exb, asb, adb, rows, semi, sema, semg = bufs[b]
                pltpu.make_async_copy(as_sh.at[srcb], asb, sema).wait()
                pltpu.make_async_copy(ad_sh.at[dstb], adb, sema).wait()
                for k in range(K // 16):
                    z = asb[pl.ds(k * 16, 16)] + adb[pl.ds(k * 16, 16)]
                    z = jnp.where(z > 0, z, z * jnp.float32(0.2))
                    exb[pl.ds(k * 16, 16)] = jnp.exp(z)
            for b in (0, 1):
                srcb, dstb, exb, asb, adb, rows, semi, sema, semg = bufs[b]
                pltpu.make_async_copy(hp_hbm.at[srcb], rows, semg).wait()

                def scale_row(r, carry2):
                    exr = plsc.load_gather(exb, [jnp.zeros((16,), jnp.int32) + r])
                    for cg in range(wp // 16):
                        rows[r, pl.ds(cg * 16, 16)] = rows[r, pl.ds(cg * 16, 16)] * exr
                    return carry2
                lax.fori_loop(0, K, scale_row, 0)
                pltpu.sync_copy(rows, acc.at[dstb], add=True)
                if issue_next:
                    off = base + (2 * i + 2 + b) * K
                    pltpu.make_async_copy(
                        src_hbm.at[pl.ds(off, K)], srcb, semi).start()
                    pltpu.make_async_copy(
                        dst_hbm.at[pl.ds(off, K)], dstb, semi).start()

        def pair_body(i, carry):
            do_pair(i, True)
            return carry
        lax.fori_loop(0, pairs - 1, pair_body, 0)
        do_pair(pairs - 1, False)

        plsc.subcore_barrier()
        done = 0
        while done < RPS:
            sz = min(K, RPS - done)
            pltpu.sync_copy(acc.at[pl.ds(row0_ + done, sz)],
                            out_hbm.at[c, pl.ds(row0_ + done, sz)])
            done += sz

    kern = pl.kernel(
        body,
        out_type=jax.ShapeDtypeStruct((NCORES, NP, wp), jnp.float32),
        mesh=mesh,
        scratch_types=[
            pltpu.VMEM((K,), jnp.int32),
            pltpu.VMEM((K,), jnp.int32),
            pltpu.VMEM((K,), jnp.int32),
            pltpu.VMEM((K,), jnp.int32),
            pltpu.VMEM((K,), jnp.float32),
            pltpu.VMEM((K,), jnp.float32),
            pltpu.VMEM((K,), jnp.float32),
            pltpu.VMEM((K,), jnp.float32),
            pltpu.VMEM((K,), jnp.float32),
            pltpu.VMEM((K,), jnp.float32),
            pltpu.VMEM((K, wp), jnp.float32),
            pltpu.VMEM((K, wp), jnp.float32),
            pltpu.VMEM_SHARED((NP,), jnp.float32),
            pltpu.VMEM_SHARED((NP,), jnp.float32),
            pltpu.VMEM_SHARED((NP, wp), jnp.float32),
            pltpu.SemaphoreType.DMA,
            pltpu.SemaphoreType.DMA,
            pltpu.SemaphoreType.DMA,
            pltpu.SemaphoreType.DMA,
            pltpu.SemaphoreType.DMA,
            pltpu.SemaphoreType.DMA,
        ],
        compiler_params=pltpu.CompilerParams(needs_layout_passes=False,
                                             use_tc_tiling_on_sc=False),
    )
    return kern(hp, src, dst, aa)


BN = 1000
NB = N // BN


def _lin1_body(x_ref, w_ref, av_ref, hp_ref, aa_ref):
    h = jnp.dot(x_ref[...], w_ref[...], preferred_element_type=jnp.float32)
    pad = (lax.broadcasted_iota(jnp.int32, (BN, 16), 1) == 0).astype(jnp.float32)
    hp_ref[...] = jnp.concatenate([h, pad], axis=1)
    asv = jnp.sum(h * av_ref[0:1, :], axis=1)
    adv = jnp.sum(h * av_ref[1:2, :], axis=1)
    aa_ref[...] = jnp.concatenate(
        [asv[None], adv[None], jnp.zeros((6, BN), jnp.float32)], axis=0)[None]


def _tc_lin1(x, w, asrc, adst):
    din, dout = w.shape
    av = jnp.zeros((8, dout), jnp.float32).at[0].set(asrc).at[1].set(adst)
    return pl.pallas_call(
        _lin1_body,
        grid=(NB,),
        in_specs=[
            pl.BlockSpec((BN, din), lambda i: (i, 0)),
            pl.BlockSpec((din, dout), lambda i: (0, 0)),
            pl.BlockSpec((8, dout), lambda i: (0, 0)),
        ],
        out_specs=[
            pl.BlockSpec((BN, dout + 16), lambda i: (i, 0)),
            pl.BlockSpec((1, 8, BN), lambda i: (i, 0, 0)),
        ],
        out_shape=[
            jax.ShapeDtypeStruct((N, dout + 16), jnp.float32),
            jax.ShapeDtypeStruct((NB, 8, BN), jnp.float32),
        ],
    )(x, w, av)


def _mid_body(acc_ref, b1_ref, w2_ref, av2_ref, hp2_ref, aa2_ref):
    a = acc_ref[0] + acc_ref[1]
    d1 = w2_ref.shape[0]
    num = a[:, :d1]
    den = a[:, d1:d1 + 1]
    x1 = num / (den + jnp.float32(1e-16)) + b1_ref[0:1, :]
    h2 = jnp.dot(x1, w2_ref[...], preferred_element_type=jnp.float32)
    pad = (lax.broadcasted_iota(jnp.int32, (BN, 16), 1) == 0).astype(jnp.float32)
    hp2_ref[...] = jnp.concatenate([h2, pad], axis=1)
    asv = jnp.sum(h2 * av2_ref[0:1, :], axis=1)
    adv = jnp.sum(h2 * av2_ref[1:2, :], axis=1)
    aa2_ref[...] = jnp.concatenate(
        [asv[None], adv[None], jnp.zeros((6, BN), jnp.float32)], axis=0)[None]


def _tc_mid(accp, b1, w2, asrc2, adst2):
    d1, d2 = w2.shape
    wp1 = accp.shape[2]
    b1p = jnp.zeros((8, d1), jnp.float32).at[0].set(b1)
    av2 = jnp.zeros((8, d2), jnp.float32).at[0].set(asrc2).at[1].set(adst2)
    return pl.pallas_call(
        _mid_body,
        grid=(NB,),
        in_specs=[
            pl.BlockSpec((NCORES, BN, wp1), lambda i: (0, i, 0)),
            pl.BlockSpec((8, d1), lambda i: (0, 0)),
            pl.BlockSpec((d1, d2), lambda i: (0, 0)),
            pl.BlockSpec((8, d2), lambda i: (0, 0)),
        ],
        out_specs=[
            pl.BlockSpec((BN, d2 + 16), lambda i: (i, 0)),
            pl.BlockSpec((1, 8, BN), lambda i: (i, 0, 0)),
        ],
        out_shape=[
            jax.ShapeDtypeStruct((N, d2 + 16), jnp.float32),
            jax.ShapeDtypeStruct((NB, 8, BN), jnp.float32),
        ],
    )(accp, b1p, w2, av2)


def _final_body(accs_ref, acct_ref, b2_ref, bs_ref, bt_ref,
                lw_ref, lb_ref, out_ref, ps, pt, cnt):
    i = pl.program_id(0)

    @pl.when(i == 0)
    def _init():
        ps[...] = jnp.zeros((G, 64), jnp.float32)
        pt[...] = jnp.zeros((G, 64), jnp.float32)
        cnt[...] = jnp.zeros((G, 128), jnp.float32)

    d2 = 64
    for acc_ref, brow, pref, ccol in ((accs_ref, 0, ps, 0), (acct_ref, 1, pt, 1)):
        a = acc_ref[0] + acc_ref[1]
        x2 = a[:, :d2] / (a[:, d2:d2 + 1] + jnp.float32(1e-16)) + b2_ref[brow:brow + 1, :]
        b = bs_ref[0] if brow == 0 else bt_ref[0]
        oh = (b == lax.broadcasted_iota(jnp.int32, (G, BN), 0)).astype(jnp.float32)
        pref[...] = pref[...] + jnp.dot(oh, x2, preferred_element_type=jnp.float32)
        cnt[:, ccol:ccol + 1] = cnt[:, ccol:ccol + 1] + jnp.sum(oh, axis=1)[:, None]

    @pl.when(i == NB - 1)
    def _fin():
        pool = (ps[...] / jnp.maximum(cnt[:, 0:1], 1.0)
                + pt[...] / jnp.maximum(cnt[:, 1:2], 1.0))
        y = jnp.dot(pool, lw_ref[...], preferred_element_type=jnp.float32) + lb_ref[0:1, :]
        out_ref[...] = jax.nn.sigmoid(y)


def _tc_final(accs, acct, b2s, b2t, xs_batch, xt_batch, lin_w, lin_b):
    b2 = jnp.zeros((8, 64), jnp.float32).at[0].set(b2s).at[1].set(b2t)
    lbp = jnp.zeros((8, C), jnp.float32).at[0].set(lin_b)
    bs3 = xs_batch.astype(jnp.int32).reshape(NB, 1, BN)
    bt3 = xt_batch.astype(jnp.int32).reshape(NB, 1, BN)
    wp2 = accs.shape[2]
    return pl.pallas_call(
        _final_body,
        grid=(NB,),
        in_specs=[
            pl.BlockSpec((NCORES, BN, wp2), lambda i: (0, i, 0)),
            pl.BlockSpec((NCORES, BN, wp2), lambda i: (0, i, 0)),
            pl.BlockSpec((8, 64), lambda i: (0, 0)),
            pl.BlockSpec((1, 1, BN), lambda i: (i, 0, 0)),
            pl.BlockSpec((1, 1, BN), lambda i: (i, 0, 0)),
            pl.BlockSpec((64, C), lambda i: (0, 0)),
            pl.BlockSpec((8, C), lambda i: (0, 0)),
        ],
        out_specs=pl.BlockSpec((G, C), lambda i: (0, 0)),
        out_shape=jax.ShapeDtypeStruct((G, C), jnp.float32),
        scratch_shapes=[
            pltpu.VMEM((G, 64), jnp.float32),
            pltpu.VMEM((G, 64), jnp.float32),
            pltpu.VMEM((G, 128), jnp.float32),
        ],
    )(accs, acct, b2, bs3, bt3, lin_w, lbp)


def _pad_edges(edge_index):
    src = edge_index[0].astype(jnp.int32)
    dst = edge_index[1].astype(jnp.int32)
    pad = E_PAD - E
    src = jnp.concatenate([src, jnp.zeros((pad,), jnp.int32)])
    dst = jnp.concatenate([dst, jnp.full((pad,), N, jnp.int32)])
    return src, dst


def _pad_aa(aa):
    aa2 = aa.transpose(1, 0, 2).reshape(8, N)
    return jnp.pad(aa2, ((0, 0), (0, NP - N)))


def _tower(x, edge_index, w1, asrc1, adst1, b1, w2, asrc2, adst2):
    src, dst = _pad_edges(edge_index)
    hp1, aa1 = _tc_lin1(x, w1, asrc1, adst1)
    acc1 = _sc_edge(hp1, src, dst, _pad_aa(aa1), w1.shape[1] + 16)
    hp2, aa2 = _tc_mid(acc1, b1, w2, asrc2, adst2)
    acc2 = _sc_edge(hp2, src, dst, _pad_aa(aa2), w2.shape[1] + 16)
    return acc2


def kernel(x_s, edge_index_s, x_t, edge_index_t, xs_batch, xt_batch,
           xa1_W, xa1_asrc, xa1_adst, xa1_b, xa2_W, xa2_asrc, xa2_adst, xa2_b,
           ya1_W, ya1_asrc, ya1_adst, ya1_b, ya2_W, ya2_asrc, ya2_adst, ya2_b,
           lin_W, lin_b):
    accs = _tower(x_s, edge_index_s, xa1_W, xa1_asrc, xa1_adst, xa1_b,
                  xa2_W, xa2_asrc, xa2_adst)
    acct = _tower(x_t, edge_index_t, ya1_W, ya1_asrc, ya1_adst, ya1_b,
                  ya2_W, ya2_asrc, ya2_adst)
    return _tc_final(accs, acct, xa2_b, ya2_b, xs_batch, xt_batch, lin_W, lin_b)


# speedup vs baseline: 1.2616x; 1.2616x over previous
import functools

import jax
import jax.numpy as jnp
from jax import lax
from jax.experimental import pallas as pl
from jax.experimental.pallas import tpu as pltpu
from jax.experimental.pallas import tpu_sc as plsc

N = 10000
E = 160000
D = 256
G = 64
C = 10

NCORES = 2
NSUB = 16
NT = NCORES * NSUB
K = 128
E_PAD = 163840
UNIT = 4096
UNITS = E_PAD // UNIT
M0 = 28
NP = 10112
RPS = NP // NSUB


def _sc_edge(hp, src, dst, aa, wp):
    mesh = plsc.VectorSubcoreMesh(core_axis_name="c", subcore_axis_name="s",
                                  num_cores=NCORES, num_subcores=NSUB)

    def body(hp_hbm, src_hbm, dst_hbm, aa_hbm, out_hbm,
             src0, dst0, src1, dst1, ex0, ex1, asb0, adb0, asb1, adb1,
             rows0, rows1, as_sh, ad_sh, acc,
             semi0, semi1, sema0, sema1, semg0, semg1):
        c = lax.axis_index("c")
        s = lax.axis_index("s")
        bufs = ((src0, dst0, ex0, asb0, adb0, rows0, semi0, sema0, semg0),
                (src1, dst1, ex1, asb1, adb1, rows1, semi1, sema1, semg1))
        pairs = jnp.where(c == 0, M0, UNITS - M0)
        base = jnp.where(c == 0, 0, M0 * UNIT) + s * (2 * K * pairs)

        def zrow(r, carry):
            for cg in range(wp // 16):
                rows0[r, pl.ds(cg * 16, 16)] = jnp.zeros((16,), jnp.float32)
            return carry
        lax.fori_loop(0, K, zrow, 0)
        row0_ = s * RPS
        done = 0
        while done < RPS:
            sz = min(K, RPS - done)
            pltpu.sync_copy(rows0.at[pl.ds(0, sz)],
                            acc.at[pl.ds(row0_ + done, sz)])
            done += sz

        @pl.when(s == 0)
        def _stage_aa():
            pltpu.sync_copy(aa_hbm.at[0], as_sh)
            pltpu.sync_copy(aa_hbm.at[1], ad_sh)
        plsc.subcore_barrier()

        for b in (0, 1):
            srcb, dstb = bufs[b][0], bufs[b][1]
            semi = bufs[b][6]
            off = base + b * K
            pltpu.make_async_copy(src_hbm.at[pl.ds(off, K)], srcb, semi).start()
            pltpu.make_async_copy(dst_hbm.at[pl.ds(off, K)], dstb, semi).start()

        def do_pair(i, issue_next):
            for b in (0, 1):
                srcb, dstb, exb, asb, adb, rows, semi, sema, semg = bufs[b]
                pltpu.make_async_copy(src_hbm.at[pl.ds(0, K)], srcb, semi).wait()
                pltpu.make_async_copy(dst_hbm.at[pl.ds(0, K)], dstb, semi).wait()
                pltpu.make_async_copy(as_sh.at[srcb], asb, sema).start()
                pltpu.make_async_copy(ad_sh.at[dstb], adb, sema).start()
                pltpu.make_async_copy(hp_hbm.at[srcb], rows, semg).start()
            for b in (0, 1):
                srcb, dstb, exb, asb, adb, rows, semi, sema, semg = bufs[b]
                pltpu.make_async_copy(as_sh.at[srcb], asb, sema).wait()
                pltpu.make_async_copy(ad_sh.at[dstb], adb, sema).wait()
                for k in range(K // 16):
                    z = asb[pl.ds(k * 16, 16)] + adb[pl.ds(k * 16, 16)]
                    z = jnp.where(z > 0, z, z * jnp.float32(0.2))
                    exb[pl.ds(k * 16, 16)] = jnp.exp(z)
            for b in (0, 1):
                srcb, dstb, exb, asb, adb, rows, semi, sema, semg = bufs[b]
                pltpu.make_async_copy(hp_hbm.at[srcb], rows, semg).wait()

                def scale_row(r, carry2):
                    exr = plsc.load_gather(exb, [jnp.zeros((16,), jnp.int32) + r])
                    for cg in range(wp // 16):
                        rows[r, pl.ds(cg * 16, 16)] = rows[r, pl.ds(cg * 16, 16)] * exr
                    return carry2
                lax.fori_loop(0, K, scale_row, 0)
                pltpu.sync_copy(rows, acc.at[dstb], add=True)
                if issue_next:
                    off = base + (2 * i + 2 + b) * K
                    pltpu.make_async_copy(
                        src_hbm.at[pl.ds(off, K)], srcb, semi).start()
                    pltpu.make_async_copy(
                        dst_hbm.at[pl.ds(off, K)], dstb, semi).start()

        def pair_body(i, carry):
            do_pair(i, True)
            return carry
        lax.fori_loop(0, pairs - 1, pair_body, 0)
        do_pair(pairs - 1, False)

        plsc.subcore_barrier()
        done = 0
        while done < RPS:
            sz = min(K, RPS - done)
            pltpu.sync_copy(acc.at[pl.ds(row0_ + done, sz)],
                            out_hbm.at[c, pl.ds(row0_ + done, sz)])
            done += sz

    kern = pl.kernel(
        body,
        out_type=jax.ShapeDtypeStruct((NCORES, NP, wp), jnp.float32),
        mesh=mesh,
        scratch_types=[
            pltpu.VMEM((K,), jnp.int32),
            pltpu.VMEM((K,), jnp.int32),
            pltpu.VMEM((K,), jnp.int32),
            pltpu.VMEM((K,), jnp.int32),
            pltpu.VMEM((K,), jnp.float32),
            pltpu.VMEM((K,), jnp.float32),
            pltpu.VMEM((K,), jnp.float32),
            pltpu.VMEM((K,), jnp.float32),
            pltpu.VMEM((K,), jnp.float32),
            pltpu.VMEM((K,), jnp.float32),
            pltpu.VMEM((K, wp), jnp.float32),
            pltpu.VMEM((K, wp), jnp.float32),
            pltpu.VMEM_SHARED((NP,), jnp.float32),
            pltpu.VMEM_SHARED((NP,), jnp.float32),
            pltpu.VMEM_SHARED((NP, wp), jnp.float32),
            pltpu.SemaphoreType.DMA,
            pltpu.SemaphoreType.DMA,
            pltpu.SemaphoreType.DMA,
            pltpu.SemaphoreType.DMA,
            pltpu.SemaphoreType.DMA,
            pltpu.SemaphoreType.DMA,
        ],
        compiler_params=pltpu.CompilerParams(needs_layout_passes=False,
                                             use_tc_tiling_on_sc=False),
    )
    return kern(hp, src, dst, aa)


BN = 1000
NB = N // BN


def _lin1_body(x_ref, w_ref, av_ref, hp_ref, aa_ref):
    h = jnp.dot(x_ref[...], w_ref[...], preferred_element_type=jnp.float32)
    pad = (lax.broadcasted_iota(jnp.int32, (BN, 16), 1) == 0).astype(jnp.float32)
    hp_ref[...] = jnp.concatenate([h, pad], axis=1)
    asv = jnp.sum(h * av_ref[0:1, :], axis=1)
    adv = jnp.sum(h * av_ref[1:2, :], axis=1)
    aa_ref[...] = jnp.concatenate(
        [asv[None], adv[None], jnp.zeros((6, BN), jnp.float32)], axis=0)[None]


def _tc_lin1(x, w, asrc, adst):
    din, dout = w.shape
    av = jnp.zeros((8, dout), jnp.float32).at[0].set(asrc).at[1].set(adst)
    return pl.pallas_call(
        _lin1_body,
        grid=(NB,),
        in_specs=[
            pl.BlockSpec((BN, din), lambda i: (i, 0)),
            pl.BlockSpec((din, dout), lambda i: (0, 0)),
            pl.BlockSpec((8, dout), lambda i: (0, 0)),
        ],
        out_specs=[
            pl.BlockSpec((BN, dout + 16), lambda i: (i, 0)),
            pl.BlockSpec((1, 8, BN), lambda i: (i, 0, 0)),
        ],
        out_shape=[
            jax.ShapeDtypeStruct((N, dout + 16), jnp.float32),
            jax.ShapeDtypeStruct((NB, 8, BN), jnp.float32),
        ],
    )(x, w, av)


def _mid_body(acc_ref, b1_ref, w2_ref, av2_ref, hp2_ref, aa2_ref):
    a = acc_ref[0] + acc_ref[1]
    d1 = w2_ref.shape[0]
    num = a[:, :d1]
    den = a[:, d1:d1 + 1]
    x1 = num / (den + jnp.float32(1e-16)) + b1_ref[0:1, :]
    h2 = jnp.dot(x1, w2_ref[...], preferred_element_type=jnp.float32)
    pad = (lax.broadcasted_iota(jnp.int32, (BN, 16), 1) == 0).astype(jnp.float32)
    hp2_ref[...] = jnp.concatenate([h2, pad], axis=1)
    asv = jnp.sum(h2 * av2_ref[0:1, :], axis=1)
    adv = jnp.sum(h2 * av2_ref[1:2, :], axis=1)
    aa2_ref[...] = jnp.concatenate(
        [asv[None], adv[None], jnp.zeros((6, BN), jnp.float32)], axis=0)[None]


def _tc_mid(accp, b1, w2, asrc2, adst2):
    d1, d2 = w2.shape
    wp1 = accp.shape[2]
    b1p = jnp.zeros((8, d1), jnp.float32).at[0].set(b1)
    av2 = jnp.zeros((8, d2), jnp.float32).at[0].set(asrc2).at[1].set(adst2)
    return pl.pallas_call(
        _mid_body,
        grid=(NB,),
        in_specs=[
            pl.BlockSpec((NCORES, BN, wp1), lambda i: (0, i, 0)),
            pl.BlockSpec((8, d1), lambda i: (0, 0)),
            pl.BlockSpec((d1, d2), lambda i: (0, 0)),
            pl.BlockSpec((8, d2), lambda i: (0, 0)),
        ],
        out_specs=[
            pl.BlockSpec((BN, d2 + 16), lambda i: (i, 0)),
            pl.BlockSpec((1, 8, BN), lambda i: (i, 0, 0)),
        ],
        out_shape=[
            jax.ShapeDtypeStruct((N, d2 + 16), jnp.float32),
            jax.ShapeDtypeStruct((NB, 8, BN), jnp.float32),
        ],
    )(accp, b1p, w2, av2)


def _final_body(accs_ref, acct_ref, b2_ref, bs_ref, bt_ref,
                lw_ref, lb_ref, out_ref, ps, pt, cnt):
    i = pl.program_id(0)

    @pl.when(i == 0)
    def _init():
        ps[...] = jnp.zeros((G, 64), jnp.float32)
        pt[...] = jnp.zeros((G, 64), jnp.float32)
        cnt[...] = jnp.zeros((G, 128), jnp.float32)

    d2 = 64
    for acc_ref, brow, pref, ccol in ((accs_ref, 0, ps, 0), (acct_ref, 1, pt, 1)):
        a = acc_ref[0] + acc_ref[1]
        x2 = a[:, :d2] / (a[:, d2:d2 + 1] + jnp.float32(1e-16)) + b2_ref[brow:brow + 1, :]
        b = bs_ref[0] if brow == 0 else bt_ref[0]
        oh = (b == lax.broadcasted_iota(jnp.int32, (G, BN), 0)).astype(jnp.float32)
        pref[...] = pref[...] + jnp.dot(oh, x2, preferred_element_type=jnp.float32)
        cnt[:, ccol:ccol + 1] = cnt[:, ccol:ccol + 1] + jnp.sum(oh, axis=1)[:, None]

    @pl.when(i == NB - 1)
    def _fin():
        pool = (ps[...] / jnp.maximum(cnt[:, 0:1], 1.0)
                + pt[...] / jnp.maximum(cnt[:, 1:2], 1.0))
        y = jnp.dot(pool, lw_ref[...], preferred_element_type=jnp.float32) + lb_ref[0:1, :]
        out_ref[...] = jax.nn.sigmoid(y)


def _tc_final(accs, acct, b2s, b2t, xs_batch, xt_batch, lin_w, lin_b):
    b2 = jnp.zeros((8, 64), jnp.float32).at[0].set(b2s).at[1].set(b2t)
    lbp = jnp.zeros((8, C), jnp.float32).at[0].set(lin_b)
    bs3 = xs_batch.astype(jnp.int32).reshape(NB, 1, BN)
    bt3 = xt_batch.astype(jnp.int32).reshape(NB, 1, BN)
    wp2 = accs.shape[2]
    return pl.pallas_call(
        _final_body,
        grid=(NB,),
        in_specs=[
            pl.BlockSpec((NCORES, BN, wp2), lambda i: (0, i, 0)),
            pl.BlockSpec((NCORES, BN, wp2), lambda i: (0, i, 0)),
            pl.BlockSpec((8, 64), lambda i: (0, 0)),
            pl.BlockSpec((1, 1, BN), lambda i: (i, 0, 0)),
            pl.BlockSpec((1, 1, BN), lambda i: (i, 0, 0)),
            pl.BlockSpec((64, C), lambda i: (0, 0)),
            pl.BlockSpec((8, C), lambda i: (0, 0)),
        ],
        out_specs=pl.BlockSpec((G, C), lambda i: (0, 0)),
        out_shape=jax.ShapeDtypeStruct((G, C), jnp.float32),
        scratch_shapes=[
            pltpu.VMEM((G, 64), jnp.float32),
            pltpu.VMEM((G, 64), jnp.float32),
            pltpu.VMEM((G, 128), jnp.float32),
        ],
    )(accs, acct, b2, bs3, bt3, lin_w, lbp)


def _pad_edges(edge_index):
    src = edge_index[0].astype(jnp.int32)
    dst = edge_index[1].astype(jnp.int32)
    pad = E_PAD - E
    src = jnp.concatenate([src, jnp.zeros((pad,), jnp.int32)])
    dst = jnp.concatenate([dst, jnp.full((pad,), N, jnp.int32)])
    return src, dst


def _pad_aa(aa):
    aa2 = aa.transpose(1, 0, 2).reshape(8, N)
    return jnp.pad(aa2, ((0, 0), (0, NP - N)))


def _tower(x, edge_index, w1, asrc1, adst1, b1, w2, asrc2, adst2):
    src, dst = _pad_edges(edge_index)
    hp1, aa1 = _tc_lin1(x, w1, asrc1, adst1)
    acc1 = _sc_edge(hp1, src, dst, _pad_aa(aa1), w1.shape[1] + 16)
    hp2, aa2 = _tc_mid(acc1, b1, w2, asrc2, adst2)
    acc2 = _sc_edge(hp2, src, dst, _pad_aa(aa2), w2.shape[1] + 16)
    return acc2


def kernel(x_s, edge_index_s, x_t, edge_index_t, xs_batch, xt_batch,
           xa1_W, xa1_asrc, xa1_adst, xa1_b, xa2_W, xa2_asrc, xa2_adst, xa2_b,
           ya1_W, ya1_asrc, ya1_adst, ya1_b, ya2_W, ya2_asrc, ya2_adst, ya2_b,
           lin_W, lin_b):
    accs = _tower(x_s, edge_index_s, xa1_W, xa1_asrc, xa1_adst, xa1_b,
                  xa2_W, xa2_asrc, xa2_adst)
    acct = _tower(x_t, edge_index_t, ya1_W, ya1_asrc, ya1_adst, ya1_b,
                  ya2_W, ya2_asrc, ya2_adst)
    return _tc_final(accs, acct, xa2_b, ya2_b, xs_batch, xt_batch, lin_W, lin_b)


# speedup vs baseline: 1.3128x; 1.0406x over previous
import functools

import jax
import jax.numpy as jnp
from jax import lax
from jax.experimental import pallas as pl
from jax.experimental.pallas import tpu as pltpu
from jax.experimental.pallas import tpu_sc as plsc

N = 10000
E = 160000
D = 256
G = 64
C = 10

NCORES = 2
NSUB = 16
NT = NCORES * NSUB
K = 128
E_PAD = 163840
UNIT = 4096
UNITS = E_PAD // UNIT
M0 = 30
NP = 10112
RPS = NP // NSUB


def _sc_edge(hp, src, dst, aa, wp):
    mesh = plsc.VectorSubcoreMesh(core_axis_name="c", subcore_axis_name="s",
                                  num_cores=NCORES, num_subcores=NSUB)

    def body(hp_hbm, src_hbm, dst_hbm, aa_hbm, out_hbm,
             src0, dst0, src1, dst1, ex0, ex1, asb0, adb0, asb1, adb1,
             rows0, rows1, as_sh, ad_sh, acc,
             semi0, semi1, sema0, sema1, semg0, semg1):
        c = lax.axis_index("c")
        s = lax.axis_index("s")
        bufs = ((src0, dst0, ex0, asb0, adb0, rows0, semi0, sema0, semg0),
                (src1, dst1, ex1, asb1, adb1, rows1, semi1, sema1, semg1))
        pairs = jnp.where(c == 0, M0, UNITS - M0)
        base = jnp.where(c == 0, 0, M0 * UNIT) + s * (2 * K * pairs)

        def zrow(r, carry):
            for cg in range(wp // 16):
                rows0[r, pl.ds(cg * 16, 16)] = jnp.zeros((16,), jnp.float32)
            return carry
        lax.fori_loop(0, K, zrow, 0)
        row0_ = s * RPS
        done = 0
        while done < RPS:
            sz = min(K, RPS - done)
            pltpu.sync_copy(rows0.at[pl.ds(0, sz)],
                            acc.at[pl.ds(row0_ + done, sz)])
            done += sz

        @pl.when(s == 0)
        def _stage_aa():
            pltpu.sync_copy(aa_hbm.at[0], as_sh)
            pltpu.sync_copy(aa_hbm.at[1], ad_sh)
        plsc.subcore_barrier()

        for b in (0, 1):
            srcb, dstb = bufs[b][0], bufs[b][1]
            semi = bufs[b][6]
            off = base + b * K
            pltpu.make_async_copy(src_hbm.at[pl.ds(off, K)], srcb, semi).start()
            pltpu.make_async_copy(dst_hbm.at[pl.ds(off, K)], dstb, semi).start()

        def do_pair(i, issue_next):
            for b in (0, 1):
                srcb, dstb, exb, asb, adb, rows, semi, sema, semg = bufs[b]
                pltpu.make_async_copy(src_hbm.at[pl.ds(0, K)], srcb, semi).wait()
                pltpu.make_async_copy(dst_hbm.at[pl.ds(0, K)], dstb, semi).wait()
                pltpu.make_async_copy(as_sh.at[srcb], asb, sema).start()
                pltpu.make_async_copy(ad_sh.at[dstb], adb, sema).start()
                pltpu.make_async_copy(hp_hbm.at[srcb], rows, semg).start()
            for b in (0, 1):
                srcb, dstb, exb, asb, adb, rows, semi, sema, semg = bufs[b]
                pltpu.make_async_copy(as_sh.at[srcb], asb, sema).wait()
                pltpu.make_async_copy(ad_sh.at[dstb], adb, sema).wait()
                for k in range(K // 16):
                    z = asb[pl.ds(k * 16, 16)] + adb[pl.ds(k * 16, 16)]
                    z = jnp.where(z > 0, z, z * jnp.float32(0.2))
                    exb[pl.ds(k * 16, 16)] = jnp.exp(z)
            for b in (0, 1):
                srcb, dstb, exb, asb, adb, rows, semi, sema, semg = bufs[b]
                pltpu.make_async_copy(hp_hbm.at[srcb], rows, semg).wait()

                def scale_row(r, carry2):
                    exr = plsc.load_gather(exb, [jnp.zeros((16,), jnp.int32) + r])
                    for cg in range(wp // 16):
                        rows[r, pl.ds(cg * 16, 16)] = rows[r, pl.ds(cg * 16, 16)] * exr
                    return carry2
                lax.fori_loop(0, K, scale_row, 0)
                pltpu.sync_copy(rows, acc.at[dstb], add=True)
                if issue_next:
                    off = base + (2 * i + 2 + b) * K
                    pltpu.make_async_copy(
                        src_hbm.at[pl.ds(off, K)], srcb, semi).start()
                    pltpu.make_async_copy(
                        dst_hbm.at[pl.ds(off, K)], dstb, semi).start()

        def pair_body(i, carry):
            do_pair(i, True)
            return carry
        lax.fori_loop(0, pairs - 1, pair_body, 0)
        do_pair(pairs - 1, False)

        plsc.subcore_barrier()
        done = 0
        while done < RPS:
            sz = min(K, RPS - done)
            pltpu.sync_copy(acc.at[pl.ds(row0_ + done, sz)],
                            out_hbm.at[c, pl.ds(row0_ + done, sz)])
            done += sz

    kern = pl.kernel(
        body,
        out_type=jax.ShapeDtypeStruct((NCORES, NP, wp), jnp.float32),
        mesh=mesh,
        scratch_types=[
            pltpu.VMEM((K,), jnp.int32),
            pltpu.VMEM((K,), jnp.int32),
            pltpu.VMEM((K,), jnp.int32),
            pltpu.VMEM((K,), jnp.int32),
            pltpu.VMEM((K,), jnp.float32),
            pltpu.VMEM((K,), jnp.float32),
            pltpu.VMEM((K,), jnp.float32),
            pltpu.VMEM((K,), jnp.float32),
            pltpu.VMEM((K,), jnp.float32),
            pltpu.VMEM((K,), jnp.float32),
            pltpu.VMEM((K, wp), jnp.float32),
            pltpu.VMEM((K, wp), jnp.float32),
            pltpu.VMEM_SHARED((NP,), jnp.float32),
            pltpu.VMEM_SHARED((NP,), jnp.float32),
            pltpu.VMEM_SHARED((NP, wp), jnp.float32),
            pltpu.SemaphoreType.DMA,
            pltpu.SemaphoreType.DMA,
            pltpu.SemaphoreType.DMA,
            pltpu.SemaphoreType.DMA,
            pltpu.SemaphoreType.DMA,
            pltpu.SemaphoreType.DMA,
        ],
        compiler_params=pltpu.CompilerParams(needs_layout_passes=False,
                                             use_tc_tiling_on_sc=False),
    )
    return kern(hp, src, dst, aa)


BN = 1000
NB = N // BN


def _lin1_body(x_ref, w_ref, av_ref, hp_ref, aa_ref):
    h = jnp.dot(x_ref[...], w_ref[...], preferred_element_type=jnp.float32)
    pad = (lax.broadcasted_iota(jnp.int32, (BN, 16), 1) == 0).astype(jnp.float32)
    hp_ref[...] = jnp.concatenate([h, pad], axis=1)
    asv = jnp.sum(h * av_ref[0:1, :], axis=1)
    adv = jnp.sum(h * av_ref[1:2, :], axis=1)
    aa_ref[...] = jnp.concatenate(
        [asv[None], adv[None], jnp.zeros((6, BN), jnp.float32)], axis=0)[None]


def _tc_lin1(x, w, asrc, adst):
    din, dout = w.shape
    av = jnp.zeros((8, dout), jnp.float32).at[0].set(asrc).at[1].set(adst)
    return pl.pallas_call(
        _lin1_body,
        grid=(NB,),
        in_specs=[
            pl.BlockSpec((BN, din), lambda i: (i, 0)),
            pl.BlockSpec((din, dout), lambda i: (0, 0)),
            pl.BlockSpec((8, dout), lambda i: (0, 0)),
        ],
        out_specs=[
            pl.BlockSpec((BN, dout + 16), lambda i: (i, 0)),
            pl.BlockSpec((1, 8, BN), lambda i: (i, 0, 0)),
        ],
        out_shape=[
            jax.ShapeDtypeStruct((N, dout + 16), jnp.float32),
            jax.ShapeDtypeStruct((NB, 8, BN), jnp.float32),
        ],
    )(x, w, av)


def _mid_body(acc_ref, b1_ref, w2_ref, av2_ref, hp2_ref, aa2_ref):
    a = acc_ref[0] + acc_ref[1]
    d1 = w2_ref.shape[0]
    num = a[:, :d1]
    den = a[:, d1:d1 + 1]
    x1 = num / (den + jnp.float32(1e-16)) + b1_ref[0:1, :]
    h2 = jnp.dot(x1, w2_ref[...], preferred_element_type=jnp.float32)
    pad = (lax.broadcasted_iota(jnp.int32, (BN, 16), 1) == 0).astype(jnp.float32)
    hp2_ref[...] = jnp.concatenate([h2, pad], axis=1)
    asv = jnp.sum(h2 * av2_ref[0:1, :], axis=1)
    adv = jnp.sum(h2 * av2_ref[1:2, :], axis=1)
    aa2_ref[...] = jnp.concatenate(
        [asv[None], adv[None], jnp.zeros((6, BN), jnp.float32)], axis=0)[None]


def _tc_mid(accp, b1, w2, asrc2, adst2):
    d1, d2 = w2.shape
    wp1 = accp.shape[2]
    b1p = jnp.zeros((8, d1), jnp.float32).at[0].set(b1)
    av2 = jnp.zeros((8, d2), jnp.float32).at[0].set(asrc2).at[1].set(adst2)
    return pl.pallas_call(
        _mid_body,
        grid=(NB,),
        in_specs=[
            pl.BlockSpec((NCORES, BN, wp1), lambda i: (0, i, 0)),
            pl.BlockSpec((8, d1), lambda i: (0, 0)),
            pl.BlockSpec((d1, d2), lambda i: (0, 0)),
            pl.BlockSpec((8, d2), lambda i: (0, 0)),
        ],
        out_specs=[
            pl.BlockSpec((BN, d2 + 16), lambda i: (i, 0)),
            pl.BlockSpec((1, 8, BN), lambda i: (i, 0, 0)),
        ],
        out_shape=[
            jax.ShapeDtypeStruct((N, d2 + 16), jnp.float32),
            jax.ShapeDtypeStruct((NB, 8, BN), jnp.float32),
        ],
    )(accp, b1p, w2, av2)


def _final_body(accs_ref, acct_ref, b2_ref, bs_ref, bt_ref,
                lw_ref, lb_ref, out_ref, ps, pt, cnt):
    i = pl.program_id(0)

    @pl.when(i == 0)
    def _init():
        ps[...] = jnp.zeros((G, 64), jnp.float32)
        pt[...] = jnp.zeros((G, 64), jnp.float32)
        cnt[...] = jnp.zeros((G, 128), jnp.float32)

    d2 = 64
    for acc_ref, brow, pref, ccol in ((accs_ref, 0, ps, 0), (acct_ref, 1, pt, 1)):
        a = acc_ref[0] + acc_ref[1]
        x2 = a[:, :d2] / (a[:, d2:d2 + 1] + jnp.float32(1e-16)) + b2_ref[brow:brow + 1, :]
        b = bs_ref[0] if brow == 0 else bt_ref[0]
        oh = (b == lax.broadcasted_iota(jnp.int32, (G, BN), 0)).astype(jnp.float32)
        pref[...] = pref[...] + jnp.dot(oh, x2, preferred_element_type=jnp.float32)
        cnt[:, ccol:ccol + 1] = cnt[:, ccol:ccol + 1] + jnp.sum(oh, axis=1)[:, None]

    @pl.when(i == NB - 1)
    def _fin():
        pool = (ps[...] / jnp.maximum(cnt[:, 0:1], 1.0)
                + pt[...] / jnp.maximum(cnt[:, 1:2], 1.0))
        y = jnp.dot(pool, lw_ref[...], preferred_element_type=jnp.float32) + lb_ref[0:1, :]
        out_ref[...] = jax.nn.sigmoid(y)


def _tc_final(accs, acct, b2s, b2t, xs_batch, xt_batch, lin_w, lin_b):
    b2 = jnp.zeros((8, 64), jnp.float32).at[0].set(b2s).at[1].set(b2t)
    lbp = jnp.zeros((8, C), jnp.float32).at[0].set(lin_b)
    bs3 = xs_batch.astype(jnp.int32).reshape(NB, 1, BN)
    bt3 = xt_batch.astype(jnp.int32).reshape(NB, 1, BN)
    wp2 = accs.shape[2]
    return pl.pallas_call(
        _final_body,
        grid=(NB,),
        in_specs=[
            pl.BlockSpec((NCORES, BN, wp2), lambda i: (0, i, 0)),
            pl.BlockSpec((NCORES, BN, wp2), lambda i: (0, i, 0)),
            pl.BlockSpec((8, 64), lambda i: (0, 0)),
            pl.BlockSpec((1, 1, BN), lambda i: (i, 0, 0)),
            pl.BlockSpec((1, 1, BN), lambda i: (i, 0, 0)),
            pl.BlockSpec((64, C), lambda i: (0, 0)),
            pl.BlockSpec((8, C), lambda i: (0, 0)),
        ],
        out_specs=pl.BlockSpec((G, C), lambda i: (0, 0)),
        out_shape=jax.ShapeDtypeStruct((G, C), jnp.float32),
        scratch_shapes=[
            pltpu.VMEM((G, 64), jnp.float32),
            pltpu.VMEM((G, 64), jnp.float32),
            pltpu.VMEM((G, 128), jnp.float32),
        ],
    )(accs, acct, b2, bs3, bt3, lin_w, lbp)


def _pad_edges(edge_index):
    src = edge_index[0].astype(jnp.int32)
    dst = edge_index[1].astype(jnp.int32)
    pad = E_PAD - E
    src = jnp.concatenate([src, jnp.zeros((pad,), jnp.int32)])
    dst = jnp.concatenate([dst, jnp.full((pad,), N, jnp.int32)])
    return src, dst


def _pad_aa(aa):
    aa2 = aa.transpose(1, 0, 2).reshape(8, N)
    return jnp.pad(aa2, ((0, 0), (0, NP - N)))


def _tower(x, edge_index, w1, asrc1, adst1, b1, w2, asrc2, adst2):
    src, dst = _pad_edges(edge_index)
    hp1, aa1 = _tc_lin1(x, w1, asrc1, adst1)
    acc1 = _sc_edge(hp1, src, dst, _pad_aa(aa1), w1.shape[1] + 16)
    hp2, aa2 = _tc_mid(acc1, b1, w2, asrc2, adst2)
    acc2 = _sc_edge(hp2, src, dst, _pad_aa(aa2), w2.shape[1] + 16)
    return acc2


def kernel(x_s, edge_index_s, x_t, edge_index_t, xs_batch, xt_batch,
           xa1_W, xa1_asrc, xa1_adst, xa1_b, xa2_W, xa2_asrc, xa2_adst, xa2_b,
           ya1_W, ya1_asrc, ya1_adst, ya1_b, ya2_W, ya2_asrc, ya2_adst, ya2_b,
           lin_W, lin_b):
    accs = _tower(x_s, edge_index_s, xa1_W, xa1_asrc, xa1_adst, xa1_b,
                  xa2_W, xa2_asrc, xa2_adst)
    acct = _tower(x_t, edge_index_t, ya1_W, ya1_asrc, ya1_adst, ya1_b,
                  ya2_W, ya2_asrc, ya2_adst)
    return _tc_final(accs, acct, xa2_b, ya2_b, xs_batch, xt_batch, lin_W, lin_b)


# speedup vs baseline: 1.3857x; 1.0555x over previous
import functools

import jax
import jax.numpy as jnp
from jax import lax
from jax.experimental import pallas as pl
from jax.experimental.pallas import tpu as pltpu
from jax.experimental.pallas import tpu_sc as plsc

N = 10000
E = 160000
D = 256
G = 64
C = 10

NCORES = 2
NSUB = 16
NT = NCORES * NSUB
K = 128
E_PAD = 163840
UNIT = 4096
UNITS = E_PAD // UNIT
M0 = 32
NP = 10112
RPS = NP // NSUB


def _sc_edge(hp, src, dst, aa, wp):
    mesh = plsc.VectorSubcoreMesh(core_axis_name="c", subcore_axis_name="s",
                                  num_cores=NCORES, num_subcores=NSUB)

    def body(hp_hbm, src_hbm, dst_hbm, aa_hbm, out_hbm,
             src0, dst0, src1, dst1, ex0, ex1, asb0, adb0, asb1, adb1,
             rows0, rows1, as_sh, ad_sh, acc,
             semi0, semi1, sema0, sema1, semg0, semg1):
        c = lax.axis_index("c")
        s = lax.axis_index("s")
        bufs = ((src0, dst0, ex0, asb0, adb0, rows0, semi0, sema0, semg0),
                (src1, dst1, ex1, asb1, adb1, rows1, semi1, sema1, semg1))
        pairs = jnp.where(c == 0, M0, UNITS - M0)
        base = jnp.where(c == 0, 0, M0 * UNIT) + s * (2 * K * pairs)

        def zrow(r, carry):
            for cg in range(wp // 16):
                rows0[r, pl.ds(cg * 16, 16)] = jnp.zeros((16,), jnp.float32)
            return carry
        lax.fori_loop(0, K, zrow, 0)
        row0_ = s * RPS
        done = 0
        while done < RPS:
            sz = min(K, RPS - done)
            pltpu.sync_copy(rows0.at[pl.ds(0, sz)],
                            acc.at[pl.ds(row0_ + done, sz)])
            done += sz

        @pl.when(s == 0)
        def _stage_aa():
            pltpu.sync_copy(aa_hbm.at[0], as_sh)
            pltpu.sync_copy(aa_hbm.at[1], ad_sh)
        plsc.subcore_barrier()

        for b in (0, 1):
            srcb, dstb = bufs[b][0], bufs[b][1]
            semi = bufs[b][6]
            off = base + b * K
            pltpu.make_async_copy(src_hbm.at[pl.ds(off, K)], srcb, semi).start()
            pltpu.make_async_copy(dst_hbm.at[pl.ds(off, K)], dstb, semi).start()

        def do_pair(i, issue_next):
            for b in (0, 1):
                srcb, dstb, exb, asb, adb, rows, semi, sema, semg = bufs[b]
                pltpu.make_async_copy(src_hbm.at[pl.ds(0, K)], srcb, semi).wait()
                pltpu.make_async_copy(dst_hbm.at[pl.ds(0, K)], dstb, semi).wait()
                pltpu.make_async_copy(as_sh.at[srcb], asb, sema).start()
                pltpu.make_async_copy(ad_sh.at[dstb], adb, sema).start()
                pltpu.make_async_copy(hp_hbm.at[srcb], rows, semg).start()
            for b in (0, 1):
                srcb, dstb, exb, asb, adb, rows, semi, sema, semg = bufs[b]
                pltpu.make_async_copy(as_sh.at[srcb], asb, sema).wait()
                pltpu.make_async_copy(ad_sh.at[dstb], adb, sema).wait()
                for k in range(K // 16):
                    z = asb[pl.ds(k * 16, 16)] + adb[pl.ds(k * 16, 16)]
                    z = jnp.where(z > 0, z, z * jnp.float32(0.2))
                    exb[pl.ds(k * 16, 16)] = jnp.exp(z)
            for b in (0, 1):
                srcb, dstb, exb, asb, adb, rows, semi, sema, semg = bufs[b]
                pltpu.make_async_copy(hp_hbm.at[srcb], rows, semg).wait()

                def scale_row(r, carry2):
                    exr = plsc.load_gather(exb, [jnp.zeros((16,), jnp.int32) + r])
                    for cg in range(wp // 16):
                        rows[r, pl.ds(cg * 16, 16)] = rows[r, pl.ds(cg * 16, 16)] * exr
                    return carry2
                lax.fori_loop(0, K, scale_row, 0)
                pltpu.sync_copy(rows, acc.at[dstb], add=True)
                if issue_next:
                    off = base + (2 * i + 2 + b) * K
                    pltpu.make_async_copy(
                        src_hbm.at[pl.ds(off, K)], srcb, semi).start()
                    pltpu.make_async_copy(
                        dst_hbm.at[pl.ds(off, K)], dstb, semi).start()

        def pair_body(i, carry):
            do_pair(i, True)
            return carry
        lax.fori_loop(0, pairs - 1, pair_body, 0)
        do_pair(pairs - 1, False)

        plsc.subcore_barrier()
        done = 0
        while done < RPS:
            sz = min(K, RPS - done)
            pltpu.sync_copy(acc.at[pl.ds(row0_ + done, sz)],
                            out_hbm.at[c, pl.ds(row0_ + done, sz)])
            done += sz

    kern = pl.kernel(
        body,
        out_type=jax.ShapeDtypeStruct((NCORES, NP, wp), jnp.float32),
        mesh=mesh,
        scratch_types=[
            pltpu.VMEM((K,), jnp.int32),
            pltpu.VMEM((K,), jnp.int32),
            pltpu.VMEM((K,), jnp.int32),
            pltpu.VMEM((K,), jnp.int32),
            pltpu.VMEM((K,), jnp.float32),
            pltpu.VMEM((K,), jnp.float32),
            pltpu.VMEM((K,), jnp.float32),
            pltpu.VMEM((K,), jnp.float32),
            pltpu.VMEM((K,), jnp.float32),
            pltpu.VMEM((K,), jnp.float32),
            pltpu.VMEM((K, wp), jnp.float32),
            pltpu.VMEM((K, wp), jnp.float32),
            pltpu.VMEM_SHARED((NP,), jnp.float32),
            pltpu.VMEM_SHARED((NP,), jnp.float32),
            pltpu.VMEM_SHARED((NP, wp), jnp.float32),
            pltpu.SemaphoreType.DMA,
            pltpu.SemaphoreType.DMA,
            pltpu.SemaphoreType.DMA,
            pltpu.SemaphoreType.DMA,
            pltpu.SemaphoreType.DMA,
            pltpu.SemaphoreType.DMA,
        ],
        compiler_params=pltpu.CompilerParams(needs_layout_passes=False,
                                             use_tc_tiling_on_sc=False),
    )
    return kern(hp, src, dst, aa)


BN = 1000
NB = N // BN


def _lin1_body(x_ref, w_ref, av_ref, hp_ref, aa_ref):
    h = jnp.dot(x_ref[...], w_ref[...], preferred_element_type=jnp.float32)
    pad = (lax.broadcasted_iota(jnp.int32, (BN, 16), 1) == 0).astype(jnp.float32)
    hp_ref[...] = jnp.concatenate([h, pad], axis=1)
    asv = jnp.sum(h * av_ref[0:1, :], axis=1)
    adv = jnp.sum(h * av_ref[1:2, :], axis=1)
    aa_ref[...] = jnp.concatenate(
        [asv[None], adv[None], jnp.zeros((6, BN), jnp.float32)], axis=0)[None]


def _tc_lin1(x, w, asrc, adst):
    din, dout = w.shape
    av = jnp.zeros((8, dout), jnp.float32).at[0].set(asrc).at[1].set(adst)
    return pl.pallas_call(
        _lin1_body,
        grid=(NB,),
        in_specs=[
            pl.BlockSpec((BN, din), lambda i: (i, 0)),
            pl.BlockSpec((din, dout), lambda i: (0, 0)),
            pl.BlockSpec((8, dout), lambda i: (0, 0)),
        ],
        out_specs=[
            pl.BlockSpec((BN, dout + 16), lambda i: (i, 0)),
            pl.BlockSpec((1, 8, BN), lambda i: (i, 0, 0)),
        ],
        out_shape=[
            jax.ShapeDtypeStruct((N, dout + 16), jnp.float32),
            jax.ShapeDtypeStruct((NB, 8, BN), jnp.float32),
        ],
    )(x, w, av)


def _mid_body(acc_ref, b1_ref, w2_ref, av2_ref, hp2_ref, aa2_ref):
    a = acc_ref[0] + acc_ref[1]
    d1 = w2_ref.shape[0]
    num = a[:, :d1]
    den = a[:, d1:d1 + 1]
    x1 = num / (den + jnp.float32(1e-16)) + b1_ref[0:1, :]
    h2 = jnp.dot(x1, w2_ref[...], preferred_element_type=jnp.float32)
    pad = (lax.broadcasted_iota(jnp.int32, (BN, 16), 1) == 0).astype(jnp.float32)
    hp2_ref[...] = jnp.concatenate([h2, pad], axis=1)
    asv = jnp.sum(h2 * av2_ref[0:1, :], axis=1)
    adv = jnp.sum(h2 * av2_ref[1:2, :], axis=1)
    aa2_ref[...] = jnp.concatenate(
        [asv[None], adv[None], jnp.zeros((6, BN), jnp.float32)], axis=0)[None]


def _tc_mid(accp, b1, w2, asrc2, adst2):
    d1, d2 = w2.shape
    wp1 = accp.shape[2]
    b1p = jnp.zeros((8, d1), jnp.float32).at[0].set(b1)
    av2 = jnp.zeros((8, d2), jnp.float32).at[0].set(asrc2).at[1].set(adst2)
    return pl.pallas_call(
        _mid_body,
        grid=(NB,),
        in_specs=[
            pl.BlockSpec((NCORES, BN, wp1), lambda i: (0, i, 0)),
            pl.BlockSpec((8, d1), lambda i: (0, 0)),
            pl.BlockSpec((d1, d2), lambda i: (0, 0)),
            pl.BlockSpec((8, d2), lambda i: (0, 0)),
        ],
        out_specs=[
            pl.BlockSpec((BN, d2 + 16), lambda i: (i, 0)),
            pl.BlockSpec((1, 8, BN), lambda i: (i, 0, 0)),
        ],
        out_shape=[
            jax.ShapeDtypeStruct((N, d2 + 16), jnp.float32),
            jax.ShapeDtypeStruct((NB, 8, BN), jnp.float32),
        ],
    )(accp, b1p, w2, av2)


def _final_body(accs_ref, acct_ref, b2_ref, bs_ref, bt_ref,
                lw_ref, lb_ref, out_ref, ps, pt, cnt):
    i = pl.program_id(0)

    @pl.when(i == 0)
    def _init():
        ps[...] = jnp.zeros((G, 64), jnp.float32)
        pt[...] = jnp.zeros((G, 64), jnp.float32)
        cnt[...] = jnp.zeros((G, 128), jnp.float32)

    d2 = 64
    for acc_ref, brow, pref, ccol in ((accs_ref, 0, ps, 0), (acct_ref, 1, pt, 1)):
        a = acc_ref[0] + acc_ref[1]
        x2 = a[:, :d2] / (a[:, d2:d2 + 1] + jnp.float32(1e-16)) + b2_ref[brow:brow + 1, :]
        b = bs_ref[0] if brow == 0 else bt_ref[0]
        oh = (b == lax.broadcasted_iota(jnp.int32, (G, BN), 0)).astype(jnp.float32)
        pref[...] = pref[...] + jnp.dot(oh, x2, preferred_element_type=jnp.float32)
        cnt[:, ccol:ccol + 1] = cnt[:, ccol:ccol + 1] + jnp.sum(oh, axis=1)[:, None]

    @pl.when(i == NB - 1)
    def _fin():
        pool = (ps[...] / jnp.maximum(cnt[:, 0:1], 1.0)
                + pt[...] / jnp.maximum(cnt[:, 1:2], 1.0))
        y = jnp.dot(pool, lw_ref[...], preferred_element_type=jnp.float32) + lb_ref[0:1, :]
        out_ref[...] = jax.nn.sigmoid(y)


def _tc_final(accs, acct, b2s, b2t, xs_batch, xt_batch, lin_w, lin_b):
    b2 = jnp.zeros((8, 64), jnp.float32).at[0].set(b2s).at[1].set(b2t)
    lbp = jnp.zeros((8, C), jnp.float32).at[0].set(lin_b)
    bs3 = xs_batch.astype(jnp.int32).reshape(NB, 1, BN)
    bt3 = xt_batch.astype(jnp.int32).reshape(NB, 1, BN)
    wp2 = accs.shape[2]
    return pl.pallas_call(
        _final_body,
        grid=(NB,),
        in_specs=[
            pl.BlockSpec((NCORES, BN, wp2), lambda i: (0, i, 0)),
            pl.BlockSpec((NCORES, BN, wp2), lambda i: (0, i, 0)),
            pl.BlockSpec((8, 64), lambda i: (0, 0)),
            pl.BlockSpec((1, 1, BN), lambda i: (i, 0, 0)),
            pl.BlockSpec((1, 1, BN), lambda i: (i, 0, 0)),
            pl.BlockSpec((64, C), lambda i: (0, 0)),
            pl.BlockSpec((8, C), lambda i: (0, 0)),
        ],
        out_specs=pl.BlockSpec((G, C), lambda i: (0, 0)),
        out_shape=jax.ShapeDtypeStruct((G, C), jnp.float32),
        scratch_shapes=[
            pltpu.VMEM((G, 64), jnp.float32),
            pltpu.VMEM((G, 64), jnp.float32),
            pltpu.VMEM((G, 128), jnp.float32),
        ],
    )(accs, acct, b2, bs3, bt3, lin_w, lbp)


def _pad_edges(edge_index):
    src = edge_index[0].astype(jnp.int32)
    dst = edge_index[1].astype(jnp.int32)
    pad = E_PAD - E
    src = jnp.concatenate([src, jnp.zeros((pad,), jnp.int32)])
    dst = jnp.concatenate([dst, jnp.full((pad,), N, jnp.int32)])
    return src, dst


def _pad_aa(aa):
    aa2 = aa.transpose(1, 0, 2).reshape(8, N)
    return jnp.pad(aa2, ((0, 0), (0, NP - N)))


def _tower(x, edge_index, w1, asrc1, adst1, b1, w2, asrc2, adst2):
    src, dst = _pad_edges(edge_index)
    hp1, aa1 = _tc_lin1(x, w1, asrc1, adst1)
    acc1 = _sc_edge(hp1, src, dst, _pad_aa(aa1), w1.shape[1] + 16)
    hp2, aa2 = _tc_mid(acc1, b1, w2, asrc2, adst2)
    acc2 = _sc_edge(hp2, src, dst, _pad_aa(aa2), w2.shape[1] + 16)
    return acc2


def kernel(x_s, edge_index_s, x_t, edge_index_t, xs_batch, xt_batch,
           xa1_W, xa1_asrc, xa1_adst, xa1_b, xa2_W, xa2_asrc, xa2_adst, xa2_b,
           ya1_W, ya1_asrc, ya1_adst, ya1_b, ya2_W, ya2_asrc, ya2_adst, ya2_b,
           lin_W, lin_b):
    accs = _tower(x_s, edge_index_s, xa1_W, xa1_asrc, xa1_adst, xa1_b,
                  xa2_W, xa2_asrc, xa2_adst)
    acct = _tower(x_t, edge_index_t, ya1_W, ya1_asrc, ya1_adst, ya1_b,
                  ya2_W, ya2_asrc, ya2_adst)
    return _tc_final(accs, acct, xa2_b, ya2_b, xs_batch, xt_batch, lin_W, lin_b)


# speedup vs baseline: 1.3943x; 1.0062x over previous
import functools

import jax
import jax.numpy as jnp
from jax import lax
from jax.experimental import pallas as pl
from jax.experimental.pallas import tpu as pltpu
from jax.experimental.pallas import tpu_sc as plsc

N = 10000
E = 160000
D = 256
G = 64
C = 10

NCORES = 2
NSUB = 16
NT = NCORES * NSUB
K = 128
E_PAD = 163840
UNIT = 4096
UNITS = E_PAD // UNIT
M0 = 34
NP = 10112
RPS = NP // NSUB


def _sc_edge(hp, src, dst, aa, wp):
    mesh = plsc.VectorSubcoreMesh(core_axis_name="c", subcore_axis_name="s",
                                  num_cores=NCORES, num_subcores=NSUB)

    def body(hp_hbm, src_hbm, dst_hbm, aa_hbm, out_hbm,
             src0, dst0, src1, dst1, ex0, ex1, asb0, adb0, asb1, adb1,
             rows0, rows1, as_sh, ad_sh, acc,
             semi0, semi1, sema0, sema1, semg0, semg1):
        c = lax.axis_index("c")
        s = lax.axis_index("s")
        bufs = ((src0, dst0, ex0, asb0, adb0, rows0, semi0, sema0, semg0),
                (src1, dst1, ex1, asb1, adb1, rows1, semi1, sema1, semg1))
        pairs = jnp.where(c == 0, M0, UNITS - M0)
        base = jnp.where(c == 0, 0, M0 * UNIT) + s * (2 * K * pairs)

        def zrow(r, carry):
            for cg in range(wp // 16):
                rows0[r, pl.ds(cg * 16, 16)] = jnp.zeros((16,), jnp.float32)
            return carry
        lax.fori_loop(0, K, zrow, 0)
        row0_ = s * RPS
        done = 0
        while done < RPS:
            sz = min(K, RPS - done)
            pltpu.sync_copy(rows0.at[pl.ds(0, sz)],
                            acc.at[pl.ds(row0_ + done, sz)])
            done += sz

        @pl.when(s == 0)
        def _stage_aa():
            pltpu.sync_copy(aa_hbm.at[0], as_sh)
            pltpu.sync_copy(aa_hbm.at[1], ad_sh)
        plsc.subcore_barrier()

        for b in (0, 1):
            srcb, dstb = bufs[b][0], bufs[b][1]
            semi = bufs[b][6]
            off = base + b * K
            pltpu.make_async_copy(src_hbm.at[pl.ds(off, K)], srcb, semi).start()
            pltpu.make_async_copy(dst_hbm.at[pl.ds(off, K)], dstb, semi).start()

        def do_pair(i, issue_next):
            for b in (0, 1):
                srcb, dstb, exb, asb, adb, rows, semi, sema, semg = bufs[b]
                pltpu.make_async_copy(src_hbm.at[pl.ds(0, K)], srcb, semi).wait()
                pltpu.make_async_copy(dst_hbm.at[pl.ds(0, K)], dstb, semi).wait()
                pltpu.make_async_copy(as_sh.at[srcb], asb, sema).start()
                pltpu.make_async_copy(ad_sh.at[dstb], adb, sema).start()
                pltpu.make_async_copy(hp_hbm.at[srcb], rows, semg).start()
            for b in (0, 1):
                srcb, dstb, exb, asb, adb, rows, semi, sema, semg = bufs[b]
                pltpu.make_async_copy(as_sh.at[srcb], asb, sema).wait()
                pltpu.make_async_copy(ad_sh.at[dstb], adb, sema).wait()
                for k in range(K // 16):
                    z = asb[pl.ds(k * 16, 16)] + adb[pl.ds(k * 16, 16)]
                    z = jnp.where(z > 0, z, z * jnp.float32(0.2))
                    exb[pl.ds(k * 16, 16)] = jnp.exp(z)
            for b in (0, 1):
                srcb, dstb, exb, asb, adb, rows, semi, sema, semg = bufs[b]
                pltpu.make_async_copy(hp_hbm.at[srcb], rows, semg).wait()

                def scale_row(r, carry2):
                    exr = plsc.load_gather(exb, [jnp.zeros((16,), jnp.int32) + r])
                    for cg in range(wp // 16):
                        rows[r, pl.ds(cg * 16, 16)] = rows[r, pl.ds(cg * 16, 16)] * exr
                    return carry2
                lax.fori_loop(0, K, scale_row, 0)
                pltpu.sync_copy(rows, acc.at[dstb], add=True)
                if issue_next:
                    off = base + (2 * i + 2 + b) * K
                    pltpu.make_async_copy(
                        src_hbm.at[pl.ds(off, K)], srcb, semi).start()
                    pltpu.make_async_copy(
                        dst_hbm.at[pl.ds(off, K)], dstb, semi).start()

        def pair_body(i, carry):
            do_pair(i, True)
            return carry
        lax.fori_loop(0, pairs - 1, pair_body, 0)
        do_pair(pairs - 1, False)

        plsc.subcore_barrier()
        done = 0
        while done < RPS:
            sz = min(K, RPS - done)
            pltpu.sync_copy(acc.at[pl.ds(row0_ + done, sz)],
                            out_hbm.at[c, pl.ds(row0_ + done, sz)])
            done += sz

    kern = pl.kernel(
        body,
        out_type=jax.ShapeDtypeStruct((NCORES, NP, wp), jnp.float32),
        mesh=mesh,
        scratch_types=[
            pltpu.VMEM((K,), jnp.int32),
            pltpu.VMEM((K,), jnp.int32),
            pltpu.VMEM((K,), jnp.int32),
            pltpu.VMEM((K,), jnp.int32),
            pltpu.VMEM((K,), jnp.float32),
            pltpu.VMEM((K,), jnp.float32),
            pltpu.VMEM((K,), jnp.float32),
            pltpu.VMEM((K,), jnp.float32),
            pltpu.VMEM((K,), jnp.float32),
            pltpu.VMEM((K,), jnp.float32),
            pltpu.VMEM((K, wp), jnp.float32),
            pltpu.VMEM((K, wp), jnp.float32),
            pltpu.VMEM_SHARED((NP,), jnp.float32),
            pltpu.VMEM_SHARED((NP,), jnp.float32),
            pltpu.VMEM_SHARED((NP, wp), jnp.float32),
            pltpu.SemaphoreType.DMA,
            pltpu.SemaphoreType.DMA,
            pltpu.SemaphoreType.DMA,
            pltpu.SemaphoreType.DMA,
            pltpu.SemaphoreType.DMA,
            pltpu.SemaphoreType.DMA,
        ],
        compiler_params=pltpu.CompilerParams(needs_layout_passes=False,
                                             use_tc_tiling_on_sc=False),
    )
    return kern(hp, src, dst, aa)


BN = 1000
NB = N // BN


def _lin1_body(x_ref, w_ref, av_ref, hp_ref, aa_ref):
    h = jnp.dot(x_ref[...], w_ref[...], preferred_element_type=jnp.float32)
    pad = (lax.broadcasted_iota(jnp.int32, (BN, 16), 1) == 0).astype(jnp.float32)
    hp_ref[...] = jnp.concatenate([h, pad], axis=1)
    asv = jnp.sum(h * av_ref[0:1, :], axis=1)
    adv = jnp.sum(h * av_ref[1:2, :], axis=1)
    aa_ref[...] = jnp.concatenate(
        [asv[None], adv[None], jnp.zeros((6, BN), jnp.float32)], axis=0)[None]


def _tc_lin1(x, w, asrc, adst):
    din, dout = w.shape
    av = jnp.zeros((8, dout), jnp.float32).at[0].set(asrc).at[1].set(adst)
    return pl.pallas_call(
        _lin1_body,
        grid=(NB,),
        in_specs=[
            pl.BlockSpec((BN, din), lambda i: (i, 0)),
            pl.BlockSpec((din, dout), lambda i: (0, 0)),
            pl.BlockSpec((8, dout), lambda i: (0, 0)),
        ],
        out_specs=[
            pl.BlockSpec((BN, dout + 16), lambda i: (i, 0)),
            pl.BlockSpec((1, 8, BN), lambda i: (i, 0, 0)),
        ],
        out_shape=[
            jax.ShapeDtypeStruct((N, dout + 16), jnp.float32),
            jax.ShapeDtypeStruct((NB, 8, BN), jnp.float32),
        ],
    )(x, w, av)


def _mid_body(acc_ref, b1_ref, w2_ref, av2_ref, hp2_ref, aa2_ref):
    a = acc_ref[0] + acc_ref[1]
    d1 = w2_ref.shape[0]
    num = a[:, :d1]
    den = a[:, d1:d1 + 1]
    x1 = num / (den + jnp.float32(1e-16)) + b1_ref[0:1, :]
    h2 = jnp.dot(x1, w2_ref[...], preferred_element_type=jnp.float32)
    pad = (lax.broadcasted_iota(jnp.int32, (BN, 16), 1) == 0).astype(jnp.float32)
    hp2_ref[...] = jnp.concatenate([h2, pad], axis=1)
    asv = jnp.sum(h2 * av2_ref[0:1, :], axis=1)
    adv = jnp.sum(h2 * av2_ref[1:2, :], axis=1)
    aa2_ref[...] = jnp.concatenate(
        [asv[None], adv[None], jnp.zeros((6, BN), jnp.float32)], axis=0)[None]


def _tc_mid(accp, b1, w2, asrc2, adst2):
    d1, d2 = w2.shape
    wp1 = accp.shape[2]
    b1p = jnp.zeros((8, d1), jnp.float32).at[0].set(b1)
    av2 = jnp.zeros((8, d2), jnp.float32).at[0].set(asrc2).at[1].set(adst2)
    return pl.pallas_call(
        _mid_body,
        grid=(NB,),
        in_specs=[
            pl.BlockSpec((NCORES, BN, wp1), lambda i: (0, i, 0)),
            pl.BlockSpec((8, d1), lambda i: (0, 0)),
            pl.BlockSpec((d1, d2), lambda i: (0, 0)),
            pl.BlockSpec((8, d2), lambda i: (0, 0)),
        ],
        out_specs=[
            pl.BlockSpec((BN, d2 + 16), lambda i: (i, 0)),
            pl.BlockSpec((1, 8, BN), lambda i: (i, 0, 0)),
        ],
        out_shape=[
            jax.ShapeDtypeStruct((N, d2 + 16), jnp.float32),
            jax.ShapeDtypeStruct((NB, 8, BN), jnp.float32),
        ],
    )(accp, b1p, w2, av2)


def _final_body(accs_ref, acct_ref, b2_ref, bs_ref, bt_ref,
                lw_ref, lb_ref, out_ref, ps, pt, cnt):
    i = pl.program_id(0)

    @pl.when(i == 0)
    def _init():
        ps[...] = jnp.zeros((G, 64), jnp.float32)
        pt[...] = jnp.zeros((G, 64), jnp.float32)
        cnt[...] = jnp.zeros((G, 128), jnp.float32)

    d2 = 64
    for acc_ref, brow, pref, ccol in ((accs_ref, 0, ps, 0), (acct_ref, 1, pt, 1)):
        a = acc_ref[0] + acc_ref[1]
        x2 = a[:, :d2] / (a[:, d2:d2 + 1] + jnp.float32(1e-16)) + b2_ref[brow:brow + 1, :]
        b = bs_ref[0] if brow == 0 else bt_ref[0]
        oh = (b == lax.broadcasted_iota(jnp.int32, (G, BN), 0)).astype(jnp.float32)
        pref[...] = pref[...] + jnp.dot(oh, x2, preferred_element_type=jnp.float32)
        cnt[:, ccol:ccol + 1] = cnt[:, ccol:ccol + 1] + jnp.sum(oh, axis=1)[:, None]

    @pl.when(i == NB - 1)
    def _fin():
        pool = (ps[...] / jnp.maximum(cnt[:, 0:1], 1.0)
                + pt[...] / jnp.maximum(cnt[:, 1:2], 1.0))
        y = jnp.dot(pool, lw_ref[...], preferred_element_type=jnp.float32) + lb_ref[0:1, :]
        out_ref[...] = jax.nn.sigmoid(y)


def _tc_final(accs, acct, b2s, b2t, xs_batch, xt_batch, lin_w, lin_b):
    b2 = jnp.zeros((8, 64), jnp.float32).at[0].set(b2s).at[1].set(b2t)
    lbp = jnp.zeros((8, C), jnp.float32).at[0].set(lin_b)
    bs3 = xs_batch.astype(jnp.int32).reshape(NB, 1, BN)
    bt3 = xt_batch.astype(jnp.int32).reshape(NB, 1, BN)
    wp2 = accs.shape[2]
    return pl.pallas_call(
        _final_body,
        grid=(NB,),
        in_specs=[
            pl.BlockSpec((NCORES, BN, wp2), lambda i: (0, i, 0)),
            pl.BlockSpec((NCORES, BN, wp2), lambda i: (0, i, 0)),
            pl.BlockSpec((8, 64), lambda i: (0, 0)),
            pl.BlockSpec((1, 1, BN), lambda i: (i, 0, 0)),
            pl.BlockSpec((1, 1, BN), lambda i: (i, 0, 0)),
            pl.BlockSpec((64, C), lambda i: (0, 0)),
            pl.BlockSpec((8, C), lambda i: (0, 0)),
        ],
        out_specs=pl.BlockSpec((G, C), lambda i: (0, 0)),
        out_shape=jax.ShapeDtypeStruct((G, C), jnp.float32),
        scratch_shapes=[
            pltpu.VMEM((G, 64), jnp.float32),
            pltpu.VMEM((G, 64), jnp.float32),
            pltpu.VMEM((G, 128), jnp.float32),
        ],
    )(accs, acct, b2, bs3, bt3, lin_w, lbp)


def _pad_edges(edge_index):
    src = edge_index[0].astype(jnp.int32)
    dst = edge_index[1].astype(jnp.int32)
    pad = E_PAD - E
    src = jnp.concatenate([src, jnp.zeros((pad,), jnp.int32)])
    dst = jnp.concatenate([dst, jnp.full((pad,), N, jnp.int32)])
    return src, dst


def _pad_aa(aa):
    aa2 = aa.transpose(1, 0, 2).reshape(8, N)
    return jnp.pad(aa2, ((0, 0), (0, NP - N)))


def _tower(x, edge_index, w1, asrc1, adst1, b1, w2, asrc2, adst2):
    src, dst = _pad_edges(edge_index)
    hp1, aa1 = _tc_lin1(x, w1, asrc1, adst1)
    acc1 = _sc_edge(hp1, src, dst, _pad_aa(aa1), w1.shape[1] + 16)
    hp2, aa2 = _tc_mid(acc1, b1, w2, asrc2, adst2)
    acc2 = _sc_edge(hp2, src, dst, _pad_aa(aa2), w2.shape[1] + 16)
    return acc2


def kernel(x_s, edge_index_s, x_t, edge_index_t, xs_batch, xt_batch,
           xa1_W, xa1_asrc, xa1_adst, xa1_b, xa2_W, xa2_asrc, xa2_adst, xa2_b,
           ya1_W, ya1_asrc, ya1_adst, ya1_b, ya2_W, ya2_asrc, ya2_adst, ya2_b,
           lin_W, lin_b):
    accs = _tower(x_s, edge_index_s, xa1_W, xa1_asrc, xa1_adst, xa1_b,
                  xa2_W, xa2_asrc, xa2_adst)
    acct = _tower(x_t, edge_index_t, ya1_W, ya1_asrc, ya1_adst, ya1_b,
                  ya2_W, ya2_asrc, ya2_adst)
    return _tc_final(accs, acct, xa2_b, ya2_b, xs_batch, xt_batch, lin_W, lin_b)
